# baseline (speedup 1.0000x reference)
import functools

import jax
import jax.numpy as jnp
from jax import lax
from jax.experimental import pallas as pl
from jax.experimental.pallas import tpu as pltpu
from jax.experimental.pallas import tpu_sc as plsc

N = 10000
E = 160000
D = 128
H = 128
HP = 144
OUT = 313

_BM = 640

_NC = 2
_NS = 16
_NW = _NC * _NS
_L = 16
_CHUNK = 128
_NCHUNK = E // _CHUNK
_JMAX = (_NCHUNK + _NW - 1) // _NW


def _sc_mesh():
    return plsc.VectorSubcoreMesh(core_axis_name="c", subcore_axis_name="s")


def _gather_pair_sum(tab, src, dst):
    @functools.partial(
        pl.kernel,
        out_type=jax.ShapeDtypeStruct((E, D), jnp.float32),
        mesh=_sc_mesh(),
        scratch_types=[
            pltpu.VMEM((_CHUNK,), jnp.int32),
            pltpu.VMEM((_CHUNK,), jnp.int32),
            pltpu.VMEM((_CHUNK, D), jnp.float32),
            pltpu.VMEM((_CHUNK, D), jnp.float32),
            pltpu.SemaphoreType.DMA,
            pltpu.SemaphoreType.DMA,
        ],
    )
    def body(tab_hbm, src_hbm, dst_hbm, out_hbm, sidx_v, didx_v, ra_v, rb_v,
             sem_a, sem_b):
        w = lax.axis_index("s") * _NC + lax.axis_index("c")

        def step(j, carry):
            c = w + _NW * j

            @pl.when(c < _NCHUNK)
            def _():
                base = c * _CHUNK
                pltpu.sync_copy(src_hbm.at[pl.ds(base, _CHUNK)], sidx_v)
                pltpu.sync_copy(dst_hbm.at[pl.ds(base, _CHUNK)], didx_v)
                cpa = pltpu.async_copy(tab_hbm.at[sidx_v], ra_v, sem_a)
                cpb = pltpu.async_copy(tab_hbm.at[didx_v], rb_v, sem_b)
                cpa.wait()
                cpb.wait()

                def add_row(r, cc):
                    for k in range(D // _L):
                        sl = pl.ds(k * _L, _L)
                        ra_v[r, sl] = ra_v[r, sl] + rb_v[r, sl]
                    return cc

                lax.fori_loop(0, _CHUNK, add_row, 0)
                pltpu.sync_copy(ra_v, out_hbm.at[pl.ds(base, _CHUNK)])

            return carry

        lax.fori_loop(0, _JMAX, step, 0)

    return body(tab, src, dst)


def _edge_mlp_body(q_ref, ea_ref, wl3a_ref, bl3_ref, wl3b_ref, wm1_ref, bm1_ref,
                   wm2_ref, bm2_ref, wl4_ref, bl4_ref, out_ref):
    za = jnp.dot(q_ref[...], wl3a_ref[...], preferred_element_type=jnp.float32)
    eb = jnp.dot(ea_ref[...], wl3b_ref[...], preferred_element_type=jnp.float32)
    u0 = jnp.maximum(za + eb + bl3_ref[...], 0.0)
    u1 = jnp.dot(u0, wm1_ref[...], preferred_element_type=jnp.float32)
    u1 = jnp.maximum(u1 + bm1_ref[...], 0.0)
    u2 = jnp.dot(u1, wm2_ref[...], preferred_element_type=jnp.float32)
    u2 = jnp.maximum(u2 + bm2_ref[...], 0.0)
    yb = jnp.dot(u2, wl4_ref[...], preferred_element_type=jnp.float32)
    out_ref[...] = yb + bl4_ref[...]


def _edge_mlp(q, ea8, wl3a, bl3p, wl3b8, wm1p, bm1p, wm2p, bm2p, wl4p, bl4p):
    grid = (E // _BM,)
    return pl.pallas_call(
        _edge_mlp_body,
        grid=grid,
        in_specs=[
            pl.BlockSpec((_BM, D), lambda i: (i, 0)),
            pl.BlockSpec((_BM, 8), lambda i: (i, 0)),
            pl.BlockSpec((D, HP), lambda i: (0, 0)),
            pl.BlockSpec((1, HP), lambda i: (0, 0)),
            pl.BlockSpec((8, HP), lambda i: (0, 0)),
            pl.BlockSpec((HP, HP), lambda i: (0, 0)),
            pl.BlockSpec((1, HP), lambda i: (0, 0)),
            pl.BlockSpec((HP, HP), lambda i: (0, 0)),
            pl.BlockSpec((1, HP), lambda i: (0, 0)),
            pl.BlockSpec((HP, OUT), lambda i: (0, 0)),
            pl.BlockSpec((1, OUT), lambda i: (0, 0)),
        ],
        out_specs=pl.BlockSpec((_BM, OUT), lambda i: (i, 0)),
        out_shape=jax.ShapeDtypeStruct((E, OUT), jnp.float32),
    )(q, ea8, wl3a, bl3p, wl3b8, wm1p, bm1p, wm2p, bm2p, wl4p, bl4p)


def _pad2(a, r, c):
    return jnp.pad(a, ((0, r - a.shape[0]), (0, c - a.shape[1])))


def kernel(x, edge_index, edge_attr, shift, W1, a1_src, a1_dst, We1, a1_edge, b1,
           W2, a2_src, a2_dst, We2, a2_edge, b2, W_l2, b_l2, W_l3, b_l3,
           Wm1, bm1, Wm2, bm2, W_l4, b_l4):
    src = edge_index[0]
    dst = edge_index[1]
    mask = src != dst
    maskf = mask.astype(jnp.float32)
    cnt = jax.ops.segment_sum(maskf, dst, num_segments=N)
    loop_attr = jax.ops.segment_sum(edge_attr * maskf[:, None], dst,
                                    num_segments=N) / jnp.maximum(cnt, 1.0)[:, None]

    def gat(xin, W, a_s, a_d, We, a_e, b):
        h = xin @ W
        asn = h @ a_s
        adn = h @ a_d
        c = We @ a_e
        ae = edge_attr @ c
        ae_loop = loop_attr @ c
        alpha = asn[src] + adn[dst] + ae
        alpha = jnp.where(alpha >= 0, alpha, 0.2 * alpha)
        ex = jnp.where(mask, jnp.exp(alpha), 0.0)
        al = asn + adn + ae_loop
        al = jnp.where(al >= 0, al, 0.2 * al)
        exl = jnp.exp(al)
        num = jax.ops.segment_sum(h[src] * ex[:, None], dst, num_segments=N)
        num = num + h * exl[:, None]
        den = jax.ops.segment_sum(ex, dst, num_segments=N) + exl
        return num / (den[:, None] + 1e-16) + b

    y0 = jax.nn.relu(gat(x, W1, a1_src, a1_dst, We1, a1_edge, b1))
    y1 = jax.nn.relu(gat(y0, W2, a2_src, a2_dst, We2, a2_edge, b2))
    y4 = jax.nn.relu((y0 + y1) @ W_l2 + b_l2)

    q = _gather_pair_sum(y4, src, dst)

    ea8 = jnp.pad(edge_attr, ((0, 0), (0, 6)))
    wl3a = jnp.pad(W_l3[:H], ((0, 0), (0, HP - (H + 2))))
    bl3p = jnp.pad(b_l3, (0, HP - (H + 2)))[None, :]
    wl3b8 = jnp.pad(W_l3[H:], ((0, 6), (0, HP - (H + 2))))
    wm1p = _pad2(Wm1, HP, HP)
    wm2p = _pad2(Wm2, HP, HP)
    wl4p = jnp.pad(W_l4, ((0, HP - (H + 2)), (0, 0)))
    bm1p = jnp.pad(bm1, (0, HP - (H + 2)))[None, :]
    bm2p = jnp.pad(bm2, (0, HP - (H + 2)))[None, :]
    bl4p = b_l4[None, :]

    return _edge_mlp(q, ea8, wl3a, bl3p, wl3b8, wm1p, bm1p, wm2p, bm2p, wl4p,
                     bl4p)


# speedup vs baseline: 3.8887x; 3.8887x over previous
import functools

import jax
import jax.numpy as jnp
from jax import lax
from jax.experimental import pallas as pl
from jax.experimental.pallas import tpu as pltpu
from jax.experimental.pallas import tpu_sc as plsc

N = 10000
E = 160000
D = 128
H = 128
HP = 144
OUT = 313

_BM = 640

_NC = 2
_NS = 16
_NW = _NC * _NS
_L = 16
_CHUNK = 128
_NCHUNK = E // _CHUNK
_JMAX = (_NCHUNK + _NW - 1) // _NW


def _sc_mesh():
    return plsc.VectorSubcoreMesh(core_axis_name="c", subcore_axis_name="s")


def _gather_pair_sum(tab, src, dst):
    @functools.partial(
        pl.kernel,
        out_type=jax.ShapeDtypeStruct((E, D), jnp.float32),
        mesh=_sc_mesh(),
        scratch_types=[
            pltpu.VMEM((_CHUNK,), jnp.int32),
            pltpu.VMEM((_CHUNK,), jnp.int32),
            pltpu.VMEM((_CHUNK, D), jnp.float32),
            pltpu.VMEM((_CHUNK, D), jnp.float32),
            pltpu.SemaphoreType.DMA,
            pltpu.SemaphoreType.DMA,
        ],
    )
    def body(tab_hbm, src_hbm, dst_hbm, out_hbm, sidx_v, didx_v, ra_v, rb_v,
             sem_a, sem_b):
        w = lax.axis_index("s") * _NC + lax.axis_index("c")

        def step(j, carry):
            c = w + _NW * j

            @pl.when(c < _NCHUNK)
            def _():
                base = c * _CHUNK
                pltpu.sync_copy(src_hbm.at[pl.ds(base, _CHUNK)], sidx_v)
                pltpu.sync_copy(dst_hbm.at[pl.ds(base, _CHUNK)], didx_v)
                cpa = pltpu.async_copy(tab_hbm.at[sidx_v], ra_v, sem_a)
                cpb = pltpu.async_copy(tab_hbm.at[didx_v], rb_v, sem_b)
                cpa.wait()
                cpb.wait()

                def add_row(r, cc):
                    for k in range(D // _L):
                        sl = pl.ds(k * _L, _L)
                        ra_v[r, sl] = ra_v[r, sl] + rb_v[r, sl]
                    return cc

                lax.fori_loop(0, _CHUNK, add_row, 0)
                pltpu.sync_copy(ra_v, out_hbm.at[pl.ds(base, _CHUNK)])

            return carry

        lax.fori_loop(0, _JMAX, step, 0)

    return body(tab, src, dst)


_N16 = 10240
_NA = 10240


def _gat_alpha_pass(asn, adn, src, dst, ea0, ea1, cvec):
    cols = _N16 // _NS

    @functools.partial(
        pl.kernel,
        out_type=(jax.ShapeDtypeStruct((E,), jnp.float32),
                  jax.ShapeDtypeStruct((_NC, _N16), jnp.float32),
                  jax.ShapeDtypeStruct((_NC, _NS, _N16), jnp.float32)),
        mesh=_sc_mesh(),
        scratch_types=[
            pltpu.VMEM((N,), jnp.float32),
            pltpu.VMEM((N,), jnp.float32),
            pltpu.VMEM((_N16,), jnp.float32),
            pltpu.VMEM((_CHUNK,), jnp.int32),
            pltpu.VMEM((_CHUNK,), jnp.int32),
            pltpu.VMEM((_CHUNK,), jnp.float32),
            pltpu.VMEM((_CHUNK,), jnp.float32),
            pltpu.VMEM((_CHUNK,), jnp.float32),
            pltpu.VMEM((16,), jnp.float32),
        ],
        compiler_params=pltpu.CompilerParams(needs_layout_passes=False),
    )
    def body(asn_hbm, adn_hbm, src_hbm, dst_hbm, ea0_hbm, ea1_hbm, cvec_hbm,
             ex_out, den_out, den_scr, asn_v, adn_v, den_v, sidx_v, didx_v,
             a0_v, a1_v, ex_v, cv_v):
        cid = lax.axis_index("c")
        sid = lax.axis_index("s")
        w = sid * _NC + cid

        pltpu.sync_copy(asn_hbm, asn_v)
        pltpu.sync_copy(adn_hbm, adn_v)
        pltpu.sync_copy(cvec_hbm, cv_v)

        def zden(k, cc):
            den_v[pl.ds(k * _L, _L)] = jnp.zeros((_L,), jnp.float32)
            return cc

        lax.fori_loop(0, _N16 // _L, zden, 0)
        cvv = cv_v[pl.ds(0, _L)]
        c0 = cvv[0]
        c1 = cvv[1]

        def step(j, carry):
            c = w + _NW * j

            @pl.when(c < _NCHUNK)
            def _():
                base = c * _CHUNK
                pltpu.sync_copy(src_hbm.at[pl.ds(base, _CHUNK)], sidx_v)
                pltpu.sync_copy(dst_hbm.at[pl.ds(base, _CHUNK)], didx_v)
                pltpu.sync_copy(ea0_hbm.at[pl.ds(base, _CHUNK)], a0_v)
                pltpu.sync_copy(ea1_hbm.at[pl.ds(base, _CHUNK)], a1_v)
                for g in range(_CHUNK // _L):
                    sl = pl.ds(g * _L, _L)
                    si = sidx_v[sl]
                    di = didx_v[sl]
                    av = plsc.load_gather(asn_v, [si])
                    bv = plsc.load_gather(adn_v, [di])
                    al = av + bv + a0_v[sl] * c0 + a1_v[sl] * c1
                    al = jnp.where(al >= 0, al, 0.2 * al)
                    exg = jnp.where(si != di, jnp.exp(al), 0.0)
                    ex_v[sl] = exg
                    plsc.addupdate_scatter(den_v, [di], exg)
                pltpu.sync_copy(ex_v, ex_out.at[pl.ds(base, _CHUNK)])

            return carry

        lax.fori_loop(0, _JMAX, step, 0)

        pltpu.sync_copy(den_v, den_scr.at[cid, sid])
        plsc.subcore_barrier()
        for t in range(_NS):
            pltpu.sync_copy(den_scr.at[cid, t, pl.ds(sid * cols, cols)],
                            den_v.at[pl.ds(t * cols, cols)])

        def dred(k, cc):
            sl = pl.ds(k * _L, _L)
            v = den_v[sl]
            for t in range(1, _NS):
                v = v + den_v[pl.ds(t * cols + k * _L, _L)]
            den_v[sl] = v
            return cc

        lax.fori_loop(0, cols // _L, dred, 0)
        pltpu.sync_copy(den_v.at[pl.ds(0, cols)],
                        den_out.at[cid, pl.ds(sid * cols, cols)])

    return body(asn, adn, src, dst, ea0, ea1, cvec)


def _gat_scatter_pass(h, src, dst, ex):
    rpt = _NA // _NS

    @functools.partial(
        pl.kernel,
        out_type=jax.ShapeDtypeStruct((_NC, _NA, D), jnp.float32),
        mesh=_sc_mesh(),
        scratch_types=[
            pltpu.VMEM((_CHUNK,), jnp.int32),
            pltpu.VMEM((_CHUNK,), jnp.int32),
            pltpu.VMEM((_CHUNK,), jnp.float32),
            pltpu.VMEM((_CHUNK, D), jnp.float32),
            pltpu.VMEM_SHARED((_NA, D), jnp.float32),
            pltpu.SemaphoreType.DMA,
        ],
    )
    def body(h_hbm, src_hbm, dst_hbm, ex_hbm, num_out, sidx_v, didx_v, ex_v,
             hrows_v, acc_sh, sem):
        cid = lax.axis_index("c")
        sid = lax.axis_index("s")
        w = sid * _NC + cid

        def zrow(r, cc):
            for k in range(D // _L):
                hrows_v[r, pl.ds(k * _L, _L)] = jnp.zeros((_L,), jnp.float32)
            return cc

        lax.fori_loop(0, _CHUNK, zrow, 0)
        for b in range(rpt // _CHUNK):
            pltpu.sync_copy(hrows_v,
                            acc_sh.at[pl.ds(sid * rpt + b * _CHUNK, _CHUNK)])
        plsc.subcore_barrier()

        def step(j, carry):
            c = w + _NW * j

            @pl.when(c < _NCHUNK)
            def _():
                base = c * _CHUNK
                pltpu.sync_copy(src_hbm.at[pl.ds(base, _CHUNK)], sidx_v)
                pltpu.sync_copy(dst_hbm.at[pl.ds(base, _CHUNK)], didx_v)
                pltpu.sync_copy(ex_hbm.at[pl.ds(base, _CHUNK)], ex_v)
                cp = pltpu.async_copy(h_hbm.at[sidx_v], hrows_v, sem)
                cp.wait()

                def scale_group(g, cc):
                    sl = pl.ds(g * _L, _L)
                    exg = ex_v[sl]
                    for i in range(_L):
                        r = g * _L + i
                        exb = jnp.full((_L,), exg[i], jnp.float32)
                        for k in range(D // _L):
                            ksl = pl.ds(k * _L, _L)
                            hrows_v[r, ksl] = hrows_v[r, ksl] * exb
                    return cc

                lax.fori_loop(0, _CHUNK // _L, scale_group, 0)
                pltpu.sync_copy(hrows_v, acc_sh.at[didx_v], add=True)

            return carry

        lax.fori_loop(0, _JMAX, step, 0)
        plsc.subcore_barrier()
        pltpu.sync_copy(acc_sh.at[pl.ds(sid * rpt, rpt)],
                        num_out.at[cid, pl.ds(sid * rpt, rpt)])

    return body(h, src, dst, ex)


def _edge_mlp_body(q_ref, ea_ref, wl3a_ref, bl3_ref, wl3b_ref, wm1_ref, bm1_ref,
                   wm2_ref, bm2_ref, wl4_ref, bl4_ref, out_ref):
    za = jnp.dot(q_ref[...], wl3a_ref[...], preferred_element_type=jnp.float32)
    eb = jnp.dot(ea_ref[...], wl3b_ref[...], preferred_element_type=jnp.float32)
    u0 = jnp.maximum(za + eb + bl3_ref[...], 0.0)
    u1 = jnp.dot(u0, wm1_ref[...], preferred_element_type=jnp.float32)
    u1 = jnp.maximum(u1 + bm1_ref[...], 0.0)
    u2 = jnp.dot(u1, wm2_ref[...], preferred_element_type=jnp.float32)
    u2 = jnp.maximum(u2 + bm2_ref[...], 0.0)
    yb = jnp.dot(u2, wl4_ref[...], preferred_element_type=jnp.float32)
    out_ref[...] = yb + bl4_ref[...]


def _edge_mlp(q, ea8, wl3a, bl3p, wl3b8, wm1p, bm1p, wm2p, bm2p, wl4p, bl4p):
    grid = (E // _BM,)
    return pl.pallas_call(
        _edge_mlp_body,
        grid=grid,
        in_specs=[
            pl.BlockSpec((_BM, D), lambda i: (i, 0)),
            pl.BlockSpec((_BM, 8), lambda i: (i, 0)),
            pl.BlockSpec((D, HP), lambda i: (0, 0)),
            pl.BlockSpec((1, HP), lambda i: (0, 0)),
            pl.BlockSpec((8, HP), lambda i: (0, 0)),
            pl.BlockSpec((HP, HP), lambda i: (0, 0)),
            pl.BlockSpec((1, HP), lambda i: (0, 0)),
            pl.BlockSpec((HP, HP), lambda i: (0, 0)),
            pl.BlockSpec((1, HP), lambda i: (0, 0)),
            pl.BlockSpec((HP, OUT), lambda i: (0, 0)),
            pl.BlockSpec((1, OUT), lambda i: (0, 0)),
        ],
        out_specs=pl.BlockSpec((_BM, OUT), lambda i: (i, 0)),
        out_shape=jax.ShapeDtypeStruct((E, OUT), jnp.float32),
    )(q, ea8, wl3a, bl3p, wl3b8, wm1p, bm1p, wm2p, bm2p, wl4p, bl4p)


def _pad2(a, r, c):
    return jnp.pad(a, ((0, r - a.shape[0]), (0, c - a.shape[1])))


def kernel(x, edge_index, edge_attr, shift, W1, a1_src, a1_dst, We1, a1_edge, b1,
           W2, a2_src, a2_dst, We2, a2_edge, b2, W_l2, b_l2, W_l3, b_l3,
           Wm1, bm1, Wm2, bm2, W_l4, b_l4):
    src = edge_index[0]
    dst = edge_index[1]
    mask = src != dst
    maskf = mask.astype(jnp.float32)
    cnt = jax.ops.segment_sum(maskf, dst, num_segments=N)
    loop_attr = jax.ops.segment_sum(edge_attr * maskf[:, None], dst,
                                    num_segments=N) / jnp.maximum(cnt, 1.0)[:, None]

    ea0 = edge_attr[:, 0]
    ea1 = edge_attr[:, 1]

    def gat(xin, W, a_s, a_d, We, a_e, b):
        h = xin @ W
        asn = h @ a_s
        adn = h @ a_d
        c = We @ a_e
        ae_loop = loop_attr @ c
        al = asn + adn + ae_loop
        al = jnp.where(al >= 0, al, 0.2 * al)
        exl = jnp.exp(al)
        cvec = jnp.pad(c, (0, 14))
        ex, denp, _ = _gat_alpha_pass(asn, adn, src, dst, ea0, ea1, cvec)
        nump = _gat_scatter_pass(h, src, dst, ex)
        num = nump[0, :N] + nump[1, :N] + h * exl[:, None]
        den = denp[0, :N] + denp[1, :N] + exl
        return num / (den[:, None] + 1e-16) + b

    y0 = jax.nn.relu(gat(x, W1, a1_src, a1_dst, We1, a1_edge, b1))
    y1 = jax.nn.relu(gat(y0, W2, a2_src, a2_dst, We2, a2_edge, b2))
    y4 = jax.nn.relu((y0 + y1) @ W_l2 + b_l2)

    q = _gather_pair_sum(y4, src, dst)

    ea8 = jnp.pad(edge_attr, ((0, 0), (0, 6)))
    wl3a = jnp.pad(W_l3[:H], ((0, 0), (0, HP - (H + 2))))
    bl3p = jnp.pad(b_l3, (0, HP - (H + 2)))[None, :]
    wl3b8 = jnp.pad(W_l3[H:], ((0, 6), (0, HP - (H + 2))))
    wm1p = _pad2(Wm1, HP, HP)
    wm2p = _pad2(Wm2, HP, HP)
    wl4p = jnp.pad(W_l4, ((0, HP - (H + 2)), (0, 0)))
    bm1p = jnp.pad(bm1, (0, HP - (H + 2)))[None, :]
    bm2p = jnp.pad(bm2, (0, HP - (H + 2)))[None, :]
    bl4p = b_l4[None, :]

    return _edge_mlp(q, ea8, wl3a, bl3p, wl3b8, wm1p, bm1p, wm2p, bm2p, wl4p,
                     bl4p)


# speedup vs baseline: 5.3205x; 1.3682x over previous
import functools

import jax
import jax.numpy as jnp
from jax import lax
from jax.experimental import pallas as pl
from jax.experimental.pallas import tpu as pltpu
from jax.experimental.pallas import tpu_sc as plsc

N = 10000
E = 160000
D = 128
H = 128
HP = 144
OUT = 313

_BM = 640

_NC = 2
_NS = 16
_NW = _NC * _NS
_L = 16
_CHUNK = 128
_NCHUNK = E // _CHUNK
_JMAX = (_NCHUNK + _NW - 1) // _NW


def _sc_mesh():
    return plsc.VectorSubcoreMesh(core_axis_name="c", subcore_axis_name="s")


def _gather_pair_sum(tab, src, dst):
    @functools.partial(
        pl.kernel,
        out_type=jax.ShapeDtypeStruct((E, D), jnp.float32),
        mesh=_sc_mesh(),
        scratch_types=[
            pltpu.VMEM((_CHUNK,), jnp.int32),
            pltpu.VMEM((_CHUNK,), jnp.int32),
            pltpu.VMEM((_CHUNK, D), jnp.float32),
            pltpu.VMEM((_CHUNK, D), jnp.float32),
            pltpu.SemaphoreType.DMA,
            pltpu.SemaphoreType.DMA,
        ],
    )
    def body(tab_hbm, src_hbm, dst_hbm, out_hbm, sidx_v, didx_v, ra_v, rb_v,
             sem_a, sem_b):
        w = lax.axis_index("s") * _NC + lax.axis_index("c")

        def step(j, carry):
            c = w + _NW * j

            @pl.when(c < _NCHUNK)
            def _():
                base = c * _CHUNK
                pltpu.sync_copy(src_hbm.at[pl.ds(base, _CHUNK)], sidx_v)
                pltpu.sync_copy(dst_hbm.at[pl.ds(base, _CHUNK)], didx_v)
                cpa = pltpu.async_copy(tab_hbm.at[sidx_v], ra_v, sem_a)
                cpb = pltpu.async_copy(tab_hbm.at[didx_v], rb_v, sem_b)
                cpa.wait()
                cpb.wait()

                def add_row(r, cc):
                    for k in range(D // _L):
                        sl = pl.ds(k * _L, _L)
                        ra_v[r, sl] = ra_v[r, sl] + rb_v[r, sl]
                    return cc

                lax.fori_loop(0, _CHUNK, add_row, 0)
                pltpu.sync_copy(ra_v, out_hbm.at[pl.ds(base, _CHUNK)])

            return carry

        lax.fori_loop(0, _JMAX, step, 0)

    return body(tab, src, dst)


_N16 = 10240
_NA = 10240


def _loop_attr_pass(src, dst, ea0, ea1):
    cols = _N16 // _NS

    @functools.partial(
        pl.kernel,
        out_type=(jax.ShapeDtypeStruct((_NC, 3 * _N16), jnp.float32),
                  jax.ShapeDtypeStruct((_NC, _NS, 3 * _N16), jnp.float32)),
        mesh=_sc_mesh(),
        scratch_types=[
            pltpu.VMEM((3 * _N16,), jnp.float32),
            pltpu.VMEM((_CHUNK,), jnp.int32),
            pltpu.VMEM((_CHUNK,), jnp.int32),
            pltpu.VMEM((_CHUNK,), jnp.float32),
            pltpu.VMEM((_CHUNK,), jnp.float32),
        ],
        compiler_params=pltpu.CompilerParams(needs_layout_passes=False),
    )
    def body(src_hbm, dst_hbm, ea0_hbm, ea1_hbm, out_hbm, scr_hbm, tab_v,
             sidx_v, didx_v, a0_v, a1_v):
        cid = lax.axis_index("c")
        sid = lax.axis_index("s")
        w = sid * _NC + cid

        def ztab(k, cc):
            tab_v[pl.ds(k * _L, _L)] = jnp.zeros((_L,), jnp.float32)
            return cc

        lax.fori_loop(0, 3 * _N16 // _L, ztab, 0)

        def step(j, carry):
            c = w + _NW * j

            @pl.when(c < _NCHUNK)
            def _():
                base = c * _CHUNK
                pltpu.sync_copy(src_hbm.at[pl.ds(base, _CHUNK)], sidx_v)
                pltpu.sync_copy(dst_hbm.at[pl.ds(base, _CHUNK)], didx_v)
                pltpu.sync_copy(ea0_hbm.at[pl.ds(base, _CHUNK)], a0_v)
                pltpu.sync_copy(ea1_hbm.at[pl.ds(base, _CHUNK)], a1_v)
                for g in range(_CHUNK // _L):
                    sl = pl.ds(g * _L, _L)
                    si = sidx_v[sl]
                    di = didx_v[sl]
                    m = (si != di).astype(jnp.float32)
                    plsc.addupdate_scatter(tab_v, [di], m)
                    plsc.addupdate_scatter(tab_v, [di + _N16], m * a0_v[sl])
                    plsc.addupdate_scatter(tab_v, [di + 2 * _N16],
                                           m * a1_v[sl])

            return carry

        lax.fori_loop(0, _JMAX, step, 0)

        pltpu.sync_copy(tab_v, scr_hbm.at[cid, sid])
        plsc.subcore_barrier()
        for q in range(3):
            for t in range(_NS):
                pltpu.sync_copy(
                    scr_hbm.at[cid, t, pl.ds(q * _N16 + sid * cols, cols)],
                    tab_v.at[pl.ds(t * cols, cols)])

            def qred(k, cc):
                sl = pl.ds(k * _L, _L)
                v = tab_v[sl]
                for t in range(1, _NS):
                    v = v + tab_v[pl.ds(t * cols + k * _L, _L)]
                tab_v[sl] = v
                return cc

            lax.fori_loop(0, cols // _L, qred, 0)
            pltpu.sync_copy(tab_v.at[pl.ds(0, cols)],
                            out_hbm.at[cid, pl.ds(q * _N16 + sid * cols,
                                                  cols)])

    return body(src, dst, ea0, ea1)


def _gat_alpha_pass(asn, adn, src, dst, ea0, ea1, cvec):
    cols = _N16 // _NS

    @functools.partial(
        pl.kernel,
        out_type=(jax.ShapeDtypeStruct((E,), jnp.float32),
                  jax.ShapeDtypeStruct((_NC, _N16), jnp.float32),
                  jax.ShapeDtypeStruct((_NC, _NS, _N16), jnp.float32)),
        mesh=_sc_mesh(),
        scratch_types=[
            pltpu.VMEM((N,), jnp.float32),
            pltpu.VMEM((N,), jnp.float32),
            pltpu.VMEM((_N16,), jnp.float32),
            pltpu.VMEM((_CHUNK,), jnp.int32),
            pltpu.VMEM((_CHUNK,), jnp.int32),
            pltpu.VMEM((_CHUNK,), jnp.float32),
            pltpu.VMEM((_CHUNK,), jnp.float32),
            pltpu.VMEM((_CHUNK,), jnp.float32),
            pltpu.VMEM((16,), jnp.float32),
        ],
        compiler_params=pltpu.CompilerParams(needs_layout_passes=False),
    )
    def body(asn_hbm, adn_hbm, src_hbm, dst_hbm, ea0_hbm, ea1_hbm, cvec_hbm,
             ex_out, den_out, den_scr, asn_v, adn_v, den_v, sidx_v, didx_v,
             a0_v, a1_v, ex_v, cv_v):
        cid = lax.axis_index("c")
        sid = lax.axis_index("s")
        w = sid * _NC + cid

        pltpu.sync_copy(asn_hbm, asn_v)
        pltpu.sync_copy(adn_hbm, adn_v)
        pltpu.sync_copy(cvec_hbm, cv_v)

        def zden(k, cc):
            den_v[pl.ds(k * _L, _L)] = jnp.zeros((_L,), jnp.float32)
            return cc

        lax.fori_loop(0, _N16 // _L, zden, 0)
        cvv = cv_v[pl.ds(0, _L)]
        c0 = cvv[0]
        c1 = cvv[1]

        def step(j, carry):
            c = w + _NW * j

            @pl.when(c < _NCHUNK)
            def _():
                base = c * _CHUNK
                pltpu.sync_copy(src_hbm.at[pl.ds(base, _CHUNK)], sidx_v)
                pltpu.sync_copy(dst_hbm.at[pl.ds(base, _CHUNK)], didx_v)
                pltpu.sync_copy(ea0_hbm.at[pl.ds(base, _CHUNK)], a0_v)
                pltpu.sync_copy(ea1_hbm.at[pl.ds(base, _CHUNK)], a1_v)
                for g in range(_CHUNK // _L):
                    sl = pl.ds(g * _L, _L)
                    si = sidx_v[sl]
                    di = didx_v[sl]
                    av = plsc.load_gather(asn_v, [si])
                    bv = plsc.load_gather(adn_v, [di])
                    al = av + bv + a0_v[sl] * c0 + a1_v[sl] * c1
                    al = jnp.where(al >= 0, al, 0.2 * al)
                    exg = jnp.where(si != di, jnp.exp(al), 0.0)
                    ex_v[sl] = exg
                    plsc.addupdate_scatter(den_v, [di], exg)
                pltpu.sync_copy(ex_v, ex_out.at[pl.ds(base, _CHUNK)])

            return carry

        lax.fori_loop(0, _JMAX, step, 0)

        pltpu.sync_copy(den_v, den_scr.at[cid, sid])
        plsc.subcore_barrier()
        for t in range(_NS):
            pltpu.sync_copy(den_scr.at[cid, t, pl.ds(sid * cols, cols)],
                            den_v.at[pl.ds(t * cols, cols)])

        def dred(k, cc):
            sl = pl.ds(k * _L, _L)
            v = den_v[sl]
            for t in range(1, _NS):
                v = v + den_v[pl.ds(t * cols + k * _L, _L)]
            den_v[sl] = v
            return cc

        lax.fori_loop(0, cols // _L, dred, 0)
        pltpu.sync_copy(den_v.at[pl.ds(0, cols)],
                        den_out.at[cid, pl.ds(sid * cols, cols)])

    return body(asn, adn, src, dst, ea0, ea1, cvec)


def _gat_scatter_pass(h, src, dst, ex):
    rpt = _NA // _NS

    @functools.partial(
        pl.kernel,
        out_type=jax.ShapeDtypeStruct((_NC, _NA, D), jnp.float32),
        mesh=_sc_mesh(),
        scratch_types=[
            pltpu.VMEM((_CHUNK,), jnp.int32),
            pltpu.VMEM((_CHUNK,), jnp.int32),
            pltpu.VMEM((_CHUNK,), jnp.float32),
            pltpu.VMEM((_CHUNK, D), jnp.float32),
            pltpu.VMEM_SHARED((_NA, D), jnp.float32),
            pltpu.SemaphoreType.DMA,
        ],
    )
    def body(h_hbm, src_hbm, dst_hbm, ex_hbm, num_out, sidx_v, didx_v, ex_v,
             hrows_v, acc_sh, sem):
        cid = lax.axis_index("c")
        sid = lax.axis_index("s")
        w = sid * _NC + cid

        def zrow(r, cc):
            for k in range(D // _L):
                hrows_v[r, pl.ds(k * _L, _L)] = jnp.zeros((_L,), jnp.float32)
            return cc

        lax.fori_loop(0, _CHUNK, zrow, 0)
        for b in range(rpt // _CHUNK):
            pltpu.sync_copy(hrows_v,
                            acc_sh.at[pl.ds(sid * rpt + b * _CHUNK, _CHUNK)])
        plsc.subcore_barrier()

        def step(j, carry):
            c = w + _NW * j

            @pl.when(c < _NCHUNK)
            def _():
                base = c * _CHUNK
                pltpu.sync_copy(src_hbm.at[pl.ds(base, _CHUNK)], sidx_v)
                pltpu.sync_copy(dst_hbm.at[pl.ds(base, _CHUNK)], didx_v)
                pltpu.sync_copy(ex_hbm.at[pl.ds(base, _CHUNK)], ex_v)
                cp = pltpu.async_copy(h_hbm.at[sidx_v], hrows_v, sem)
                cp.wait()

                def scale_group(g, cc):
                    sl = pl.ds(g * _L, _L)
                    exg = ex_v[sl]
                    for i in range(_L):
                        r = g * _L + i
                        exb = jnp.full((_L,), exg[i], jnp.float32)
                        for k in range(D // _L):
                            ksl = pl.ds(k * _L, _L)
                            hrows_v[r, ksl] = hrows_v[r, ksl] * exb
                    return cc

                lax.fori_loop(0, _CHUNK // _L, scale_group, 0)
                pltpu.sync_copy(hrows_v, acc_sh.at[didx_v], add=True)

            return carry

        lax.fori_loop(0, _JMAX, step, 0)
        plsc.subcore_barrier()
        pltpu.sync_copy(acc_sh.at[pl.ds(sid * rpt, rpt)],
                        num_out.at[cid, pl.ds(sid * rpt, rpt)])

    return body(h, src, dst, ex)


def _edge_mlp_body(q_ref, ea_ref, wl3a_ref, bl3_ref, wl3b_ref, wm1_ref, bm1_ref,
                   wm2_ref, bm2_ref, wl4_ref, bl4_ref, out_ref):
    za = jnp.dot(q_ref[...], wl3a_ref[...], preferred_element_type=jnp.float32)
    eb = jnp.dot(ea_ref[...], wl3b_ref[...], preferred_element_type=jnp.float32)
    u0 = jnp.maximum(za + eb + bl3_ref[...], 0.0)
    u1 = jnp.dot(u0, wm1_ref[...], preferred_element_type=jnp.float32)
    u1 = jnp.maximum(u1 + bm1_ref[...], 0.0)
    u2 = jnp.dot(u1, wm2_ref[...], preferred_element_type=jnp.float32)
    u2 = jnp.maximum(u2 + bm2_ref[...], 0.0)
    yb = jnp.dot(u2, wl4_ref[...], preferred_element_type=jnp.float32)
    out_ref[...] = yb + bl4_ref[...]


def _edge_mlp(q, ea8, wl3a, bl3p, wl3b8, wm1p, bm1p, wm2p, bm2p, wl4p, bl4p):
    grid = (E // _BM,)
    return pl.pallas_call(
        _edge_mlp_body,
        grid=grid,
        in_specs=[
            pl.BlockSpec((_BM, D), lambda i: (i, 0)),
            pl.BlockSpec((_BM, 8), lambda i: (i, 0)),
            pl.BlockSpec((D, HP), lambda i: (0, 0)),
            pl.BlockSpec((1, HP), lambda i: (0, 0)),
            pl.BlockSpec((8, HP), lambda i: (0, 0)),
            pl.BlockSpec((HP, HP), lambda i: (0, 0)),
            pl.BlockSpec((1, HP), lambda i: (0, 0)),
            pl.BlockSpec((HP, HP), lambda i: (0, 0)),
            pl.BlockSpec((1, HP), lambda i: (0, 0)),
            pl.BlockSpec((HP, OUT), lambda i: (0, 0)),
            pl.BlockSpec((1, OUT), lambda i: (0, 0)),
        ],
        out_specs=pl.BlockSpec((_BM, OUT), lambda i: (i, 0)),
        out_shape=jax.ShapeDtypeStruct((E, OUT), jnp.float32),
    )(q, ea8, wl3a, bl3p, wl3b8, wm1p, bm1p, wm2p, bm2p, wl4p, bl4p)


def _pad2(a, r, c):
    return jnp.pad(a, ((0, r - a.shape[0]), (0, c - a.shape[1])))


def kernel(x, edge_index, edge_attr, shift, W1, a1_src, a1_dst, We1, a1_edge, b1,
           W2, a2_src, a2_dst, We2, a2_edge, b2, W_l2, b_l2, W_l3, b_l3,
           Wm1, bm1, Wm2, bm2, W_l4, b_l4):
    src = edge_index[0]
    dst = edge_index[1]
    ea0 = edge_attr[:, 0]
    ea1 = edge_attr[:, 1]
    la, _ = _loop_attr_pass(src, dst, ea0, ea1)
    las = la[0] + la[1]
    cnt = las[:N]
    lsum = jnp.stack([las[_N16:_N16 + N], las[2 * _N16:2 * _N16 + N]], axis=1)
    loop_attr = lsum / jnp.maximum(cnt, 1.0)[:, None]

    def gat(xin, W, a_s, a_d, We, a_e, b):
        h = xin @ W
        asn = h @ a_s
        adn = h @ a_d
        c = We @ a_e
        ae_loop = loop_attr @ c
        al = asn + adn + ae_loop
        al = jnp.where(al >= 0, al, 0.2 * al)
        exl = jnp.exp(al)
        cvec = jnp.pad(c, (0, 14))
        ex, denp, _ = _gat_alpha_pass(asn, adn, src, dst, ea0, ea1, cvec)
        nump = _gat_scatter_pass(h, src, dst, ex)
        num = nump[0, :N] + nump[1, :N] + h * exl[:, None]
        den = denp[0, :N] + denp[1, :N] + exl
        return num / (den[:, None] + 1e-16) + b

    y0 = jax.nn.relu(gat(x, W1, a1_src, a1_dst, We1, a1_edge, b1))
    y1 = jax.nn.relu(gat(y0, W2, a2_src, a2_dst, We2, a2_edge, b2))
    y4 = jax.nn.relu((y0 + y1) @ W_l2 + b_l2)

    q = _gather_pair_sum(y4, src, dst)

    ea8 = jnp.pad(edge_attr, ((0, 0), (0, 6)))
    wl3a = jnp.pad(W_l3[:H], ((0, 0), (0, HP - (H + 2))))
    bl3p = jnp.pad(b_l3, (0, HP - (H + 2)))[None, :]
    wl3b8 = jnp.pad(W_l3[H:], ((0, 6), (0, HP - (H + 2))))
    wm1p = _pad2(Wm1, HP, HP)
    wm2p = _pad2(Wm2, HP, HP)
    wl4p = jnp.pad(W_l4, ((0, HP - (H + 2)), (0, 0)))
    bm1p = jnp.pad(bm1, (0, HP - (H + 2)))[None, :]
    bm2p = jnp.pad(bm2, (0, HP - (H + 2)))[None, :]
    bl4p = b_l4[None, :]

    return _edge_mlp(q, ea8, wl3a, bl3p, wl3b8, wm1p, bm1p, wm2p, bm2p, wl4p,
                     bl4p)
